# Initial kernel scaffold; baseline (speedup 1.0000x reference)
import jax
import jax.numpy as jnp
from jax.experimental import pallas as pl


def kernel(x, W_E):
    raise NotImplementedError("write your pallas kernel here")


# speedup vs baseline: 5.1266x; 5.1266x over previous
import functools

import jax
import jax.numpy as jnp
from jax import lax
from jax.experimental import pallas as pl
from jax.experimental.pallas import tpu as pltpu
from jax.experimental.pallas import tpu_sc as plsc

D_VOCAB = 100000
N_VARS = 4
D_VAR = 16
D_MODEL = N_VARS * D_VAR

NC, NS = 2, 16
NW = NC * NS
VB = 512
CHUNK = 512


def _softmax_t_block(w_ref, out_ref):
    X = w_ref[...]
    ys = []
    for g in range(N_VARS):
        sub = X[g * D_VAR:(g + 1) * D_VAR, :]
        m = jnp.max(sub, axis=0, keepdims=True)
        e = jnp.exp(sub - m)
        s = jnp.sum(e, axis=0, keepdims=True)
        ys.append(e / s)
    out_ref[...] = jnp.concatenate(ys, axis=0).T


def _softmax_table(W_E):
    return pl.pallas_call(
        _softmax_t_block,
        grid=(pl.cdiv(D_VOCAB, VB),),
        in_specs=[pl.BlockSpec((D_MODEL, VB), lambda i: (0, i))],
        out_specs=pl.BlockSpec((VB, D_MODEL), lambda i: (i, 0)),
        out_shape=jax.ShapeDtypeStruct((D_VOCAB, D_MODEL), jnp.float32),
    )(W_E)


@functools.lru_cache(maxsize=None)
def _make_gather(n_rows):
    b_per_w = n_rows // NW
    n_chunks = b_per_w // CHUNK
    mesh = plsc.VectorSubcoreMesh(core_axis_name="c", subcore_axis_name="s")

    @functools.partial(
        pl.kernel, mesh=mesh,
        compiler_params=pltpu.CompilerParams(use_tc_tiling_on_sc=False),
        out_type=jax.ShapeDtypeStruct((n_rows, D_MODEL), jnp.float32),
        scratch_types=[
            pltpu.VMEM((CHUNK,), jnp.int32),
            pltpu.VMEM((CHUNK, D_MODEL), jnp.float32),
            pltpu.SemaphoreType.DMA,
        ],
    )
    def gather(table_hbm, idx_hbm, out_hbm, idx_v, rows_v, sem):
        wid = lax.axis_index("s") * NC + lax.axis_index("c")
        base = wid * b_per_w

        def step(c, carry):
            off = pl.multiple_of(base + c * CHUNK, CHUNK)
            pltpu.sync_copy(idx_hbm.at[pl.ds(off, CHUNK)], idx_v)
            pltpu.async_copy(table_hbm.at[idx_v], rows_v, sem).wait()
            pltpu.sync_copy(rows_v, out_hbm.at[pl.ds(off, CHUNK)])
            return carry

        lax.fori_loop(0, n_chunks, step, 0)

    return gather


def kernel(x, W_E):
    B, H = x.shape
    n = B * H
    idx = x.reshape(n).astype(jnp.int32)
    table = _softmax_table(W_E)
    out = _make_gather(n)(table, idx)
    return out.reshape(B, H, D_MODEL)


# speedup vs baseline: 5.4703x; 1.0670x over previous
import functools

import jax
import jax.numpy as jnp
from jax import lax
from jax.experimental import pallas as pl
from jax.experimental.pallas import tpu as pltpu
from jax.experimental.pallas import tpu_sc as plsc

D_VOCAB = 100000
N_VARS = 4
D_VAR = 16
D_MODEL = N_VARS * D_VAR

NC, NS = 2, 16
NW = NC * NS
VB = 512
CHUNK = 512


def _softmax_t_block(w_ref, out_ref):
    X = w_ref[...]
    ys = []
    for g in range(N_VARS):
        sub = X[g * D_VAR:(g + 1) * D_VAR, :]
        m = jnp.max(sub, axis=0, keepdims=True)
        e = jnp.exp(sub - m)
        s = jnp.sum(e, axis=0, keepdims=True)
        ys.append(e / s)
    out_ref[...] = jnp.concatenate(ys, axis=0).T


def _softmax_table(W_E):
    return pl.pallas_call(
        _softmax_t_block,
        grid=(pl.cdiv(D_VOCAB, VB),),
        in_specs=[pl.BlockSpec((D_MODEL, VB), lambda i: (0, i))],
        out_specs=pl.BlockSpec((VB, D_MODEL), lambda i: (i, 0)),
        out_shape=jax.ShapeDtypeStruct((D_VOCAB, D_MODEL), jnp.float32),
    )(W_E)


N_BUF = 2


@functools.lru_cache(maxsize=None)
def _make_gather(n_rows):
    b_per_w = n_rows // NW
    n_chunks = b_per_w // CHUNK
    n_pairs = n_chunks // N_BUF
    mesh = plsc.VectorSubcoreMesh(core_axis_name="c", subcore_axis_name="s")

    @functools.partial(
        pl.kernel, mesh=mesh,
        compiler_params=pltpu.CompilerParams(use_tc_tiling_on_sc=False),
        out_type=jax.ShapeDtypeStruct((n_rows, D_MODEL), jnp.float32),
        scratch_types=[
            pltpu.VMEM((n_chunks, CHUNK), jnp.int32),
            pltpu.VMEM((N_BUF, CHUNK, D_MODEL), jnp.float32),
            pltpu.SemaphoreType.DMA,
            pltpu.SemaphoreType.DMA,
            pltpu.SemaphoreType.DMA,
            pltpu.SemaphoreType.DMA,
        ],
    )
    def gather(table_hbm, idx_hbm, out_hbm, idx_v, rows_v, g0, g1, o0, o1):
        wid = lax.axis_index("s") * NC + lax.axis_index("c")
        base = wid * b_per_w
        gsems = (g0, g1)
        osems = (o0, o1)

        pltpu.sync_copy(idx_hbm.at[wid], idx_v)

        def start_gather(c, b):
            pltpu.async_copy(table_hbm.at[idx_v.at[c]], rows_v.at[b], gsems[b])

        def start_out(c, b):
            off = pl.multiple_of(base, CHUNK) + c * CHUNK
            pltpu.async_copy(rows_v.at[b], out_hbm.at[pl.ds(off, CHUNK)],
                             osems[b])

        for b in range(N_BUF):
            start_gather(b, b)

        def pair(p, carry):
            for b in range(N_BUF):
                c = p * N_BUF + b
                pltpu.make_async_copy(table_hbm.at[idx_v.at[c]],
                                      rows_v.at[b], gsems[b]).wait()
                start_out(c, b)
                nxt = c + N_BUF

                @pl.when(nxt < n_chunks)
                def _():
                    pltpu.make_async_copy(
                        rows_v.at[b],
                        out_hbm.at[pl.ds(pl.multiple_of(base, CHUNK)
                                         + c * CHUNK, CHUNK)],
                        osems[b]).wait()
                    start_gather(nxt, b)

            return carry

        lax.fori_loop(0, n_pairs, pair, 0)
        for b in range(N_BUF):
            c = n_chunks - N_BUF + b
            pltpu.make_async_copy(
                rows_v.at[b],
                out_hbm.at[pl.ds(pl.multiple_of(base, CHUNK) + c * CHUNK,
                                 CHUNK)],
                osems[b]).wait()

    return gather


def kernel(x, W_E):
    B, H = x.shape
    n = B * H
    idx = x.reshape(NW, n // NW // CHUNK, CHUNK).astype(jnp.int32)
    table = _softmax_table(W_E)
    out = _make_gather(n)(table, idx)
    return out.reshape(B, H, D_MODEL)
